# baseline (speedup 1.0000x reference)
import jax
import jax.numpy as jnp
from jax.experimental import pallas as pl
from jax.experimental.pallas import tpu as pltpu

_NCORES = 2
_BN = 1000


def _partial_kernel(t_ref, f_ref, o_ref, cnt_ref):
    j = pl.program_id(1)

    @pl.when(j == 0)
    def _init():
        o_ref[...] = jnp.zeros_like(o_ref)
        cnt_ref[...] = jnp.zeros_like(cnt_ref)

    t = t_ref[...]
    f = f_ref[...]
    o_ref[0] += jax.lax.dot_general(
        t, f, (((0,), (0,)), ((), ())), preferred_element_type=jnp.float32
    )
    cnt_ref[0] += jnp.sum(t, axis=0, keepdims=True)


def _combine_kernel(s_ref, cnt_ref, o_ref):
    c = o_ref.shape[0]
    cnt = cnt_ref[0, 0] + cnt_ref[1, 0]
    eye = (
        jax.lax.broadcasted_iota(jnp.int32, (c, c), 0)
        == jax.lax.broadcasted_iota(jnp.int32, (c, c), 1)
    ).astype(jnp.float32)
    cnt_col = jax.lax.dot_general(
        eye, cnt[None], (((1,), (1,)), ((), ())),
        preferred_element_type=jnp.float32,
    )
    o_ref[...] = (s_ref[0] + s_ref[1]) / cnt_col


def kernel(features, targets):
    n, d = features.shape
    _, c = targets.shape
    steps = n // (_NCORES * _BN)
    grid = (_NCORES, steps)
    sums, counts = pl.pallas_call(
        _partial_kernel,
        grid=grid,
        in_specs=[
            pl.BlockSpec((_BN, c), lambda i, j, s=steps: (i * s + j, 0)),
            pl.BlockSpec((_BN, d), lambda i, j, s=steps: (i * s + j, 0)),
        ],
        out_specs=[
            pl.BlockSpec((1, c, d), lambda i, j: (i, 0, 0)),
            pl.BlockSpec((1, 1, c), lambda i, j: (i, 0, 0)),
        ],
        out_shape=[
            jax.ShapeDtypeStruct((_NCORES, c, d), jnp.float32),
            jax.ShapeDtypeStruct((_NCORES, 1, c), jnp.float32),
        ],
        compiler_params=pltpu.CompilerParams(
            dimension_semantics=("parallel", "arbitrary"),
        ),
    )(targets, features)
    return pl.pallas_call(
        _combine_kernel,
        out_shape=jax.ShapeDtypeStruct((c, d), jnp.float32),
    )(sums, counts)


# speedup vs baseline: 1.2331x; 1.2331x over previous
import jax
import jax.numpy as jnp
from jax.experimental import pallas as pl
from jax.experimental.pallas import tpu as pltpu

_BN = 5000


def _cc_kernel(t_ref, f_ref, o_ref, cnt_ref):
    i = pl.program_id(0)

    @pl.when(i == 0)
    def _init():
        o_ref[...] = jnp.zeros_like(o_ref)
        cnt_ref[...] = jnp.zeros_like(cnt_ref)

    t = t_ref[...]
    f = f_ref[...]
    o_ref[...] += jax.lax.dot_general(
        t, f, (((0,), (0,)), ((), ())), preferred_element_type=jnp.float32
    )
    cnt_ref[...] += jnp.sum(t, axis=0, keepdims=True)

    @pl.when(i == pl.num_programs(0) - 1)
    def _finish():
        c = o_ref.shape[0]
        eye = (
            jax.lax.broadcasted_iota(jnp.int32, (c, c), 0)
            == jax.lax.broadcasted_iota(jnp.int32, (c, c), 1)
        ).astype(jnp.float32)
        cnt_col = jax.lax.dot_general(
            eye, cnt_ref[...], (((1,), (1,)), ((), ())),
            preferred_element_type=jnp.float32,
        )
        o_ref[...] = o_ref[...] / cnt_col


def kernel(features, targets):
    n, d = features.shape
    _, c = targets.shape
    grid = (n // _BN,)
    return pl.pallas_call(
        _cc_kernel,
        grid=grid,
        in_specs=[
            pl.BlockSpec((_BN, c), lambda i: (i, 0)),
            pl.BlockSpec((_BN, d), lambda i: (i, 0)),
        ],
        out_specs=pl.BlockSpec((c, d), lambda i: (0, 0)),
        out_shape=jax.ShapeDtypeStruct((c, d), jnp.float32),
        scratch_shapes=[pltpu.VMEM((1, c), jnp.float32)],
    )(targets, features)
